# baseline (speedup 1.0000x reference)
import functools

import jax
import jax.numpy as jnp
import numpy as np
from jax import lax
from jax.experimental import pallas as pl
from jax.experimental.pallas import tpu as pltpu
from jax.experimental.pallas import tpu_sc as plsc

RATIO = 0.75


@functools.lru_cache(maxsize=None)
def _make_gather(num_rows, C, NC, NS, sizes, depth, idx_pad):
    NW = NC * NS
    n_chunks = len(sizes)
    offs = [0]
    for s in sizes:
        offs.append(offs[-1] + s)
    rows_per_w = offs[-1]
    maxc = max(sizes)
    mesh = plsc.VectorSubcoreMesh(core_axis_name="c", subcore_axis_name="s")

    @functools.partial(
        pl.kernel,
        mesh=mesh,
        out_type=jax.ShapeDtypeStruct((num_rows, C), jnp.float32),
        scratch_types=[pltpu.VMEM((idx_pad,), jnp.int32)]
        + [pltpu.VMEM((maxc, C), jnp.float32) for _ in range(depth)]
        + [pltpu.SemaphoreType.DMA, pltpu.SemaphoreType.DMA],
    )
    def gather_k(table_hbm, idx_hbm, out_hbm, idx_v, *rest):
        bufs = rest[:depth]
        gsem, ssem = rest[depth], rest[depth + 1]
        wid = lax.axis_index("s") * NC + lax.axis_index("c")
        pltpu.sync_copy(idx_hbm.at[wid], idx_v)
        base = wid * rows_per_w
        gathers = [None] * n_chunks
        scatters = [None] * n_chunks

        def gissue(c):
            return pltpu.async_copy(
                table_hbm.at[idx_v.at[pl.ds(offs[c], sizes[c])]],
                bufs[c % depth].at[pl.ds(0, sizes[c])],
                gsem,
            )

        for c in range(min(depth, n_chunks)):
            gathers[c] = gissue(c)
        for c in range(n_chunks):
            gathers[c].wait()
            scatters[c] = pltpu.async_copy(
                bufs[c % depth].at[pl.ds(0, sizes[c])],
                out_hbm.at[pl.ds(base + offs[c], sizes[c])],
                ssem,
            )
            nxt = c + depth
            if nxt < n_chunks:
                scatters[nxt - depth].wait()
                gathers[nxt] = gissue(nxt)
        for c in range(max(0, n_chunks - depth), n_chunks):
            scatters[c].wait()

    return gather_k


@functools.lru_cache(maxsize=None)
def _make_split_fb(T, B):
    def split_k(fb_ref, fwd_ref, bwd_ref):
        fwd_ref[...] = fb_ref[:, :B]
        bwd_ref[...] = fb_ref[:, B:]

    return pl.pallas_call(
        split_k,
        out_shape=(
            jax.ShapeDtypeStruct((T, B), jnp.int32),
            jax.ShapeDtypeStruct((T, B), jnp.int32),
        ),
    )


@functools.lru_cache(maxsize=None)
def _perm_indexes(T, B):
    with jax.ensure_compile_time_eval():
        perm_key = jax.random.key(42)
        keys = jax.random.split(perm_key, B)
        fwd = jnp.stack([jax.random.permutation(k, T) for k in keys], axis=-1)
        bwd = jnp.argsort(fwd, axis=0)
        return np.asarray(fwd), np.asarray(bwd)


def kernel(patches):
    T, B, C = patches.shape
    remain_T = int(T * (1 - RATIO))
    fwd_np, bwd_np = _perm_indexes(T, B)

    src_np = fwd_np[:remain_T] * B + np.arange(B, dtype=np.int32)[None, :]
    num_rows = remain_T * B

    info = plsc.get_sparse_core_info()
    NC, NS = info.num_cores, info.num_subcores
    NW = NC * NS
    rows_per_w = num_rows // NW
    assert rows_per_w * NW == num_rows
    sizes = (16, 32, 48, 48, 48, 48, 32, 16)
    depth = 3
    assert sum(sizes) == rows_per_w

    idx_pad = -(-rows_per_w // 128) * 128
    idx_np = np.zeros((NW, idx_pad), dtype=np.int32)
    idx_np[:, :rows_per_w] = src_np.reshape(NW, rows_per_w)
    idx2 = jnp.asarray(idx_np)
    table = patches.reshape(T * B, C)
    out_flat = _make_gather(
        num_rows, C, NC, NS, sizes, depth, idx_pad
    )(table, idx2)
    fb = jnp.asarray(np.concatenate([fwd_np, bwd_np], axis=1).astype(np.int32))
    fwd, bwd = _make_split_fb(T, B)(fb)
    return out_flat.reshape(remain_T, B, C), fwd, bwd


# speedup vs baseline: 1.0060x; 1.0060x over previous
import functools

import jax
import jax.numpy as jnp
import numpy as np
from jax import lax
from jax.experimental import pallas as pl
from jax.experimental.pallas import tpu as pltpu
from jax.experimental.pallas import tpu_sc as plsc

RATIO = 0.75


@functools.lru_cache(maxsize=None)
def _make_gather(num_rows, C, NC, NS, sizes, depth, idx_pad):
    NW = NC * NS
    n_chunks = len(sizes)
    offs = [0]
    for s in sizes:
        offs.append(offs[-1] + s)
    rows_per_w = offs[-1]
    maxc = max(sizes)
    mesh = plsc.VectorSubcoreMesh(core_axis_name="c", subcore_axis_name="s")

    @functools.partial(
        pl.kernel,
        mesh=mesh,
        out_type=jax.ShapeDtypeStruct((num_rows, C), jnp.float32),
        scratch_types=[pltpu.VMEM((idx_pad,), jnp.int32)]
        + [pltpu.VMEM((maxc, C), jnp.float32) for _ in range(depth)]
        + [pltpu.SemaphoreType.DMA, pltpu.SemaphoreType.DMA],
    )
    def gather_k(table_hbm, idx_hbm, out_hbm, idx_v, *rest):
        bufs = rest[:depth]
        gsem, ssem = rest[depth], rest[depth + 1]
        wid = lax.axis_index("s") * NC + lax.axis_index("c")
        pltpu.sync_copy(idx_hbm.at[wid], idx_v)
        base = wid * rows_per_w
        gathers = [None] * n_chunks
        scatters = [None] * n_chunks

        def gissue(c):
            return pltpu.async_copy(
                table_hbm.at[idx_v.at[pl.ds(offs[c], sizes[c])]],
                bufs[c % depth].at[pl.ds(0, sizes[c])],
                gsem,
            )

        for c in range(min(depth, n_chunks)):
            gathers[c] = gissue(c)
        for c in range(n_chunks):
            gathers[c].wait()
            scatters[c] = pltpu.async_copy(
                bufs[c % depth].at[pl.ds(0, sizes[c])],
                out_hbm.at[pl.ds(base + offs[c], sizes[c])],
                ssem,
            )
            nxt = c + depth
            if nxt < n_chunks:
                scatters[nxt - depth].wait()
                gathers[nxt] = gissue(nxt)
        for c in range(max(0, n_chunks - depth), n_chunks):
            scatters[c].wait()

    return gather_k


@functools.lru_cache(maxsize=None)
def _make_split_fb(T, B):
    def split_k(fb_ref, fwd_ref, bwd_ref):
        fwd_ref[...] = fb_ref[:, :B]
        bwd_ref[...] = fb_ref[:, B:]

    return pl.pallas_call(
        split_k,
        out_shape=(
            jax.ShapeDtypeStruct((T, B), jnp.int32),
            jax.ShapeDtypeStruct((T, B), jnp.int32),
        ),
    )


@functools.lru_cache(maxsize=None)
def _perm_indexes(T, B):
    with jax.ensure_compile_time_eval():
        perm_key = jax.random.key(42)
        keys = jax.random.split(perm_key, B)
        fwd = jnp.stack([jax.random.permutation(k, T) for k in keys], axis=-1)
        bwd = jnp.argsort(fwd, axis=0)
        return np.asarray(fwd), np.asarray(bwd)


def kernel(patches):
    T, B, C = patches.shape
    remain_T = int(T * (1 - RATIO))
    fwd_np, bwd_np = _perm_indexes(T, B)

    src_np = fwd_np[:remain_T] * B + np.arange(B, dtype=np.int32)[None, :]
    num_rows = remain_T * B

    info = plsc.get_sparse_core_info()
    NC, NS = info.num_cores, info.num_subcores
    NW = NC * NS
    rows_per_w = num_rows // NW
    assert rows_per_w * NW == num_rows
    sizes = (48,) * 6
    depth = 3
    assert sum(sizes) == rows_per_w

    idx_pad = -(-rows_per_w // 128) * 128
    idx_np = np.zeros((NW, idx_pad), dtype=np.int32)
    idx_np[:, :rows_per_w] = src_np.reshape(NW, rows_per_w)
    idx2 = jnp.asarray(idx_np)
    table = patches.reshape(T * B, C)
    out_flat = _make_gather(
        num_rows, C, NC, NS, sizes, depth, idx_pad
    )(table, idx2)
    fb = jnp.asarray(np.concatenate([fwd_np, bwd_np], axis=1).astype(np.int32))
    fwd, bwd = _make_split_fb(T, B)(fb)
    return out_flat.reshape(remain_T, B, C), fwd, bwd
